# baseline (speedup 1.0000x reference)
import functools

import jax
import jax.numpy as jnp
from jax import lax
from jax.experimental import pallas as pl
from jax.experimental.pallas import tpu as pltpu
from jax.experimental.pallas import tpu_sc as plsc

B, H, W, C = 8, 384, 384, 96
HB = 96
NH = H // HB
K = 9
PAIRS = B * C
CHUNK = 16
NCHUNKS = PAIRS // CHUNK
ROWS_PER_CHUNK = CHUNK * 3
TAPS_PER_CHUNK = CHUNK * K
FLAT = ROWS_PER_CHUNK * W
NW = 32
NS = 16


def _peaks_body(x_ref, rows3_ref, lofs_ref, mask_ref, gx_ref, gy_ref,
                cmax_acc, ybest_val, ybest_idx):
    b = pl.program_id(0)
    h = pl.program_id(1)
    x = x_ref[0]

    colmax = jnp.max(x, axis=0)

    @pl.when(h == 0)
    def _():
        cmax_acc[...] = colmax

    @pl.when(h > 0)
    def _():
        cmax_acc[...] = jnp.maximum(cmax_acc[...], colmax)

    bval = jnp.max(colmax, axis=1, keepdims=True)
    hio = lax.broadcasted_iota(jnp.int32, (HB, C, W), 0)
    cand = jnp.where(x == bval[None], hio, HB)
    bidx = jnp.min(jnp.min(cand, axis=0), axis=1,
                   keepdims=True) + h * HB

    @pl.when(h == 0)
    def _():
        ybest_val[...] = bval
        ybest_idx[...] = bidx

    @pl.when(h > 0)
    def _():
        upd = bval > ybest_val[...]
        ybest_idx[...] = jnp.where(upd, bidx, ybest_idx[...])
        ybest_val[...] = jnp.maximum(ybest_val[...], bval)

    @pl.when(h == NH - 1)
    def _():
        cm = cmax_acc[...]
        xval = jnp.max(cm, axis=1, keepdims=True)
        wio = lax.broadcasted_iota(jnp.int32, (C, W), 1)
        px = jnp.min(jnp.where(cm == xval, wio, W),
                     axis=1, keepdims=True)
        py = ybest_idx[...]

        dy3 = lax.broadcasted_iota(jnp.int32, (C, 3), 1) - 1
        cio3 = lax.broadcasted_iota(jnp.int32, (C, 3), 0)
        yy3c = jnp.clip(py + dy3, 0, H - 1)
        rows3_ref[0] = (b * H + yy3c) * C + cio3

        kk = lax.broadcasted_iota(jnp.int32, (C, K), 1)
        cio = lax.broadcasted_iota(jnp.int32, (C, K), 0)
        dy = kk // 3 - 1
        dx = kk % 3 - 1
        yy = py + dy
        xx = px + dx
        valid = (yy >= 0) & (yy < H) & (xx >= 0) & (xx < W)
        lofs_ref[0] = (((cio % CHUNK) * 3 + (kk // 3)) * W
                       + jnp.clip(xx, 0, W - 1))
        mask_ref[0] = valid.astype(jnp.float32)
        gx_ref[0] = xx.astype(jnp.float32)
        gy_ref[0] = yy.astype(jnp.float32)


def _peaks_call(pt):
    return pl.pallas_call(
        _peaks_body,
        grid=(B, NH),
        in_specs=[pl.BlockSpec((1, HB, C, W), lambda b, h: (b, h, 0, 0))],
        out_specs=([pl.BlockSpec((1, C, 3), lambda b, h: (b, 0, 0))]
                   + [pl.BlockSpec((1, C, K), lambda b, h: (b, 0, 0))] * 4),
        out_shape=[
            jax.ShapeDtypeStruct((B, C, 3), jnp.int32),
            jax.ShapeDtypeStruct((B, C, K), jnp.int32),
            jax.ShapeDtypeStruct((B, C, K), jnp.float32),
            jax.ShapeDtypeStruct((B, C, K), jnp.float32),
            jax.ShapeDtypeStruct((B, C, K), jnp.float32),
        ],
        scratch_shapes=[
            pltpu.VMEM((C, W), jnp.float32),
            pltpu.VMEM((C, 1), jnp.float32),
            pltpu.VMEM((C, 1), jnp.int32),
        ],
        compiler_params=pltpu.CompilerParams(
            dimension_semantics=("parallel", "arbitrary")),
    )(pt)


def _make_patch_kernel():
    mesh = plsc.VectorSubcoreMesh(core_axis_name="c", subcore_axis_name="s")

    @functools.partial(
        pl.kernel,
        mesh=mesh,
        out_type=[
            jax.ShapeDtypeStruct((PAIRS,), jnp.float32),
            jax.ShapeDtypeStruct((PAIRS,), jnp.float32),
        ],
        scratch_types=[
            pltpu.VMEM((ROWS_PER_CHUNK,), jnp.int32),
            pltpu.VMEM((ROWS_PER_CHUNK, W), jnp.float32),
            pltpu.VMEM((FLAT,), jnp.float32),
            pltpu.VMEM((K, CHUNK), jnp.int32),
            pltpu.VMEM((TAPS_PER_CHUNK,), jnp.int32),
            pltpu.VMEM((TAPS_PER_CHUNK,), jnp.float32),
            pltpu.VMEM((K, CHUNK), jnp.float32),
            pltpu.VMEM((K, CHUNK), jnp.float32),
            pltpu.VMEM((K, CHUNK), jnp.float32),
            pltpu.VMEM((CHUNK,), jnp.float32),
            pltpu.VMEM((CHUNK,), jnp.float32),
            pltpu.VMEM_SHARED((NS * FLAT,), jnp.float32),
            pltpu.SemaphoreType.DMA,
        ],
    )
    def patch_kernel(table_hbm, rows3_hbm, lofs_hbm, mask_hbm, gx_hbm, gy_hbm,
                     outx_hbm, outy_hbm,
                     ridx_v, rows_v, flat_v, lofs_v, idx_v, vals_v,
                     mask_v, gx_v, gy_v, outx_v, outy_v, shared, sem):
        sid = lax.axis_index("s")
        wid = sid * 2 + lax.axis_index("c")
        sbase = sid * FLAT

        def do_chunk(j):
            base = j * CHUNK
            pltpu.sync_copy(
                rows3_hbm.at[pl.ds(j * ROWS_PER_CHUNK, ROWS_PER_CHUNK)],
                ridx_v)
            pltpu.async_copy(table_hbm.at[ridx_v], rows_v, sem).wait()
            for r in range(ROWS_PER_CHUNK):
                for t in range(W // CHUNK):
                    flat_v[pl.ds(r * W + t * CHUNK, CHUNK)] = (
                        rows_v[r, pl.ds(t * CHUNK, CHUNK)])
            pltpu.sync_copy(flat_v, shared.at[pl.ds(sbase, FLAT)])
            pltpu.sync_copy(lofs_hbm.at[j], lofs_v)
            pltpu.sync_copy(mask_hbm.at[j], mask_v)
            pltpu.sync_copy(gx_hbm.at[j], gx_v)
            pltpu.sync_copy(gy_hbm.at[j], gy_v)
            for k in range(K):
                idx_v[pl.ds(k * CHUNK, CHUNK)] = lofs_v[k] + sbase
            pltpu.async_copy(shared.at[idx_v], vals_v, sem).wait()
            s = xacc = yacc = None
            for k in range(K):
                v = vals_v[pl.ds(k * CHUNK, CHUNK)] * mask_v[k]
                s = v if s is None else s + v
                xv = v * gx_v[k]
                yv = v * gy_v[k]
                xacc = xv if xacc is None else xacc + xv
                yacc = yv if yacc is None else yacc + yv
            outx_v[...] = (xacc / s - (W // 2)) * (1.0 / W)
            outy_v[...] = (yacc / s - (H // 2)) * (1.0 / H)
            pltpu.sync_copy(outx_v, outx_hbm.at[pl.ds(base, CHUNK)])
            pltpu.sync_copy(outy_v, outy_hbm.at[pl.ds(base, CHUNK)])

        do_chunk(wid)

        @pl.when(wid + NW < NCHUNKS)
        def _():
            do_chunk(wid + NW)

    return patch_kernel


_patch_kernel_cache = []


def _get_patch_kernel():
    if not _patch_kernel_cache:
        _patch_kernel_cache.append(_make_patch_kernel())
    return _patch_kernel_cache[0]


def kernel(pred):
    assert pred.shape == (B, H, W, C)
    pt = jnp.transpose(pred, (0, 1, 3, 2))
    rows3, lofs, mask, gx, gy = _peaks_call(pt)

    def to_chunks(a):
        return a.reshape(NCHUNKS, CHUNK, K).transpose(0, 2, 1)

    rows3_flat = rows3.reshape(-1)
    table = pt.reshape(B * H * C, W)
    outx, outy = _get_patch_kernel()(table, rows3_flat, to_chunks(lofs),
                                     to_chunks(mask), to_chunks(gx),
                                     to_chunks(gy))
    return jnp.stack([outx.reshape(B, C), outy.reshape(B, C)], axis=-1)


# speedup vs baseline: 1.0329x; 1.0329x over previous
import functools

import jax
import jax.numpy as jnp
from jax import lax
from jax.experimental import pallas as pl
from jax.experimental.pallas import tpu as pltpu
from jax.experimental.pallas import tpu_sc as plsc

B, H, W, C = 8, 384, 384, 96
HB = 128
NH = H // HB
K = 9
PAIRS = B * C
VL = 16
CHUNK = 32
NCHUNKS = PAIRS // CHUNK
ROWS_PER_CHUNK = CHUNK * 3
TAPS_PER_CHUNK = CHUNK * K
FLAT = ROWS_PER_CHUNK * W
NW = 32
NS = 16


def _peaks_body(x_ref, rows3_ref, lofs_ref, mask_ref, gx_ref, gy_ref,
                cmax_acc, ybest_val, ybest_idx):
    b = pl.program_id(0)
    h = pl.program_id(1)
    x = x_ref[0]

    colmax = jnp.max(x, axis=0)

    @pl.when(h == 0)
    def _():
        cmax_acc[...] = colmax

    @pl.when(h > 0)
    def _():
        cmax_acc[...] = jnp.maximum(cmax_acc[...], colmax)

    bval = jnp.max(colmax, axis=1, keepdims=True)
    hio = lax.broadcasted_iota(jnp.int32, (HB, C, W), 0)
    cand = jnp.where(x == bval[None], hio, HB)
    bidx = jnp.min(jnp.min(cand, axis=0), axis=1,
                   keepdims=True) + h * HB

    @pl.when(h == 0)
    def _():
        ybest_val[...] = bval
        ybest_idx[...] = bidx

    @pl.when(h > 0)
    def _():
        upd = bval > ybest_val[...]
        ybest_idx[...] = jnp.where(upd, bidx, ybest_idx[...])
        ybest_val[...] = jnp.maximum(ybest_val[...], bval)

    @pl.when(h == NH - 1)
    def _():
        cm = cmax_acc[...]
        xval = jnp.max(cm, axis=1, keepdims=True)
        wio = lax.broadcasted_iota(jnp.int32, (C, W), 1)
        px = jnp.min(jnp.where(cm == xval, wio, W),
                     axis=1, keepdims=True)
        py = ybest_idx[...]

        dy3 = lax.broadcasted_iota(jnp.int32, (C, 3), 1) - 1
        cio3 = lax.broadcasted_iota(jnp.int32, (C, 3), 0)
        yy3c = jnp.clip(py + dy3, 0, H - 1)
        rows3_ref[0] = (b * H + yy3c) * C + cio3

        kk = lax.broadcasted_iota(jnp.int32, (C, K), 1)
        cio = lax.broadcasted_iota(jnp.int32, (C, K), 0)
        dy = kk // 3 - 1
        dx = kk % 3 - 1
        yy = py + dy
        xx = px + dx
        valid = (yy >= 0) & (yy < H) & (xx >= 0) & (xx < W)
        lofs_ref[0] = (((cio % CHUNK) * 3 + (kk // 3)) * W
                       + jnp.clip(xx, 0, W - 1))
        mask_ref[0] = valid.astype(jnp.float32)
        gx_ref[0] = xx.astype(jnp.float32)
        gy_ref[0] = yy.astype(jnp.float32)


def _peaks_call(pt):
    return pl.pallas_call(
        _peaks_body,
        grid=(B, NH),
        in_specs=[pl.BlockSpec((1, HB, C, W), lambda b, h: (b, h, 0, 0))],
        out_specs=([pl.BlockSpec((1, C, 3), lambda b, h: (b, 0, 0))]
                   + [pl.BlockSpec((1, C, K), lambda b, h: (b, 0, 0))] * 4),
        out_shape=[
            jax.ShapeDtypeStruct((B, C, 3), jnp.int32),
            jax.ShapeDtypeStruct((B, C, K), jnp.int32),
            jax.ShapeDtypeStruct((B, C, K), jnp.float32),
            jax.ShapeDtypeStruct((B, C, K), jnp.float32),
            jax.ShapeDtypeStruct((B, C, K), jnp.float32),
        ],
        scratch_shapes=[
            pltpu.VMEM((C, W), jnp.float32),
            pltpu.VMEM((C, 1), jnp.float32),
            pltpu.VMEM((C, 1), jnp.int32),
        ],
        compiler_params=pltpu.CompilerParams(
            dimension_semantics=("parallel", "arbitrary")),
    )(pt)


def _make_patch_kernel():
    mesh = plsc.VectorSubcoreMesh(core_axis_name="c", subcore_axis_name="s")

    @functools.partial(
        pl.kernel,
        mesh=mesh,
        out_type=[
            jax.ShapeDtypeStruct((PAIRS,), jnp.float32),
            jax.ShapeDtypeStruct((PAIRS,), jnp.float32),
        ],
        scratch_types=[
            pltpu.VMEM((ROWS_PER_CHUNK,), jnp.int32),
            pltpu.VMEM((ROWS_PER_CHUNK, W), jnp.float32),
            pltpu.VMEM((FLAT,), jnp.float32),
            pltpu.VMEM((K, CHUNK), jnp.int32),
            pltpu.VMEM((TAPS_PER_CHUNK,), jnp.int32),
            pltpu.VMEM((TAPS_PER_CHUNK,), jnp.float32),
            pltpu.VMEM((K, CHUNK), jnp.float32),
            pltpu.VMEM((K, CHUNK), jnp.float32),
            pltpu.VMEM((K, CHUNK), jnp.float32),
            pltpu.VMEM((CHUNK,), jnp.float32),
            pltpu.VMEM((CHUNK,), jnp.float32),
            pltpu.VMEM_SHARED((NS * FLAT,), jnp.float32),
            pltpu.SemaphoreType.DMA,
        ],
    )
    def patch_kernel(table_hbm, rows3_hbm, lofs_hbm, mask_hbm, gx_hbm, gy_hbm,
                     outx_hbm, outy_hbm,
                     ridx_v, rows_v, flat_v, lofs_v, idx_v, vals_v,
                     mask_v, gx_v, gy_v, outx_v, outy_v, shared, sem):
        sid = lax.axis_index("s")
        wid = sid * 2 + lax.axis_index("c")
        sbase = sid * FLAT

        @pl.when(wid < NCHUNKS)
        def _():
            j = wid
            base = j * CHUNK
            pltpu.sync_copy(
                rows3_hbm.at[pl.ds(j * ROWS_PER_CHUNK, ROWS_PER_CHUNK)],
                ridx_v)
            pltpu.async_copy(table_hbm.at[ridx_v], rows_v, sem).wait()
            for r in range(ROWS_PER_CHUNK):
                for t in range(W // VL):
                    flat_v[pl.ds(r * W + t * VL, VL)] = (
                        rows_v[r, pl.ds(t * VL, VL)])
            pltpu.sync_copy(flat_v, shared.at[pl.ds(sbase, FLAT)])
            pltpu.sync_copy(lofs_hbm.at[j], lofs_v)
            pltpu.sync_copy(mask_hbm.at[j], mask_v)
            pltpu.sync_copy(gx_hbm.at[j], gx_v)
            pltpu.sync_copy(gy_hbm.at[j], gy_v)
            for k in range(K):
                for g in range(CHUNK // VL):
                    idx_v[pl.ds(k * CHUNK + g * VL, VL)] = (
                        lofs_v[k, pl.ds(g * VL, VL)] + sbase)
            pltpu.async_copy(shared.at[idx_v], vals_v, sem).wait()
            for g in range(CHUNK // VL):
                gs = pl.ds(g * VL, VL)
                s = xacc = yacc = None
                for k in range(K):
                    v = (vals_v[pl.ds(k * CHUNK + g * VL, VL)]
                         * mask_v[k, gs])
                    s = v if s is None else s + v
                    xv = v * gx_v[k, gs]
                    yv = v * gy_v[k, gs]
                    xacc = xv if xacc is None else xacc + xv
                    yacc = yv if yacc is None else yacc + yv
                outx_v[gs] = (xacc / s - (W // 2)) * (1.0 / W)
                outy_v[gs] = (yacc / s - (H // 2)) * (1.0 / H)
            pltpu.sync_copy(outx_v, outx_hbm.at[pl.ds(base, CHUNK)])
            pltpu.sync_copy(outy_v, outy_hbm.at[pl.ds(base, CHUNK)])

    return patch_kernel


_patch_kernel_cache = []


def _get_patch_kernel():
    if not _patch_kernel_cache:
        _patch_kernel_cache.append(_make_patch_kernel())
    return _patch_kernel_cache[0]


def kernel(pred):
    assert pred.shape == (B, H, W, C)
    pt = jnp.transpose(pred, (0, 1, 3, 2))
    rows3, lofs, mask, gx, gy = _peaks_call(pt)

    def to_chunks(a):
        return a.reshape(NCHUNKS, CHUNK, K).transpose(0, 2, 1)

    rows3_flat = rows3.reshape(-1)
    table = pt.reshape(B * H * C, W)
    outx, outy = _get_patch_kernel()(table, rows3_flat, to_chunks(lofs),
                                     to_chunks(mask), to_chunks(gx),
                                     to_chunks(gy))
    return jnp.stack([outx.reshape(B, C), outy.reshape(B, C)], axis=-1)
